# baseline (speedup 1.0000x reference)
import functools

import jax
import jax.numpy as jnp
from jax import lax
from jax.experimental import pallas as pl
from jax.experimental.pallas import tpu as pltpu
from jax.experimental.pallas import tpu_sc as plsc

NC, NS = 2, 16
NW = NC * NS


def _sc_gather(table, idx_flat, C=128):
    RT = idx_flat.shape[0]
    F = table.shape[1]
    rows_per_w = RT // NW
    chunks_per_w = rows_per_w // C
    mesh = plsc.VectorSubcoreMesh(core_axis_name="c", subcore_axis_name="s")

    @functools.partial(
        pl.kernel, mesh=mesh,
        out_type=jax.ShapeDtypeStruct((RT, F), table.dtype),
        scratch_types=[
            pltpu.VMEM((C,), jnp.int32),
            pltpu.VMEM((C, F), table.dtype),
            pltpu.SemaphoreType.DMA,
        ],
    )
    def gk(table_hbm, idx_hbm, out_hbm, idx_v, rows_v, sem):
        wid = lax.axis_index("s") * NC + lax.axis_index("c")
        base = wid * rows_per_w

        @pl.loop(0, chunks_per_w)
        def _(ci):
            off = base + ci * C
            pltpu.sync_copy(idx_hbm.at[pl.ds(off, C)], idx_v)
            pltpu.async_copy(table_hbm.at[idx_v], rows_v, sem).wait()
            pltpu.sync_copy(rows_v, out_hbm.at[pl.ds(off, C)])

    return gk(table, idx_flat)


def _combine_block(nf_ref, rbf_ref, w1_ref, b1_ref, w2_ref, b2_ref,
                   wf_ref, agg_ref, att_ref, *, TN, K, G, F):
    R = TN * K
    f32 = jnp.float32
    rbf = rbf_ref[...].reshape(R, G)
    nrm = jnp.sqrt(jnp.sum(rbf * rbf, axis=-1, keepdims=True))
    rbfn = rbf / (nrm + 1e-8)
    h = jnp.tanh(jnp.dot(rbfn.astype(jnp.bfloat16), w1_ref[...],
                         preferred_element_type=f32) + b1_ref[...])
    cf = jnp.dot(h.astype(jnp.bfloat16), w2_ref[...],
                 preferred_element_type=f32) + b2_ref[...]

    x = nf_ref[...]
    lo = lax.bitcast_convert_type(x << 16, f32)
    hi = lax.bitcast_convert_type(x & jnp.int32(-65536), f32)
    nf = jnp.concatenate([lo, hi], axis=-1)
    conv = nf * cf
    lgcol = jnp.sum(conv * wf_ref[...], axis=-1, keepdims=True)

    jmodk = jax.lax.broadcasted_iota(jnp.int32, (R, K), 0) % K
    m_sel = (jmodk == jax.lax.broadcasted_iota(jnp.int32, (R, K), 1)).astype(f32)
    s_rows = jax.lax.broadcasted_iota(jnp.int32, (TN, R), 1) // K
    s_sum = (s_rows == jax.lax.broadcasted_iota(jnp.int32, (TN, R), 0)).astype(f32)
    e_rows = jax.lax.broadcasted_iota(jnp.int32, (R, TN), 0) // K
    s_exp = (e_rows == jax.lax.broadcasted_iota(jnp.int32, (R, TN), 1)).astype(f32)

    lg = jnp.dot(s_sum, m_sel * lgcol, preferred_element_type=f32)
    mx = jnp.max(lg, axis=-1, keepdims=True)
    ex = jnp.exp(lg - mx)
    att = ex / jnp.sum(ex, axis=-1, keepdims=True)
    att_ref[...] = att

    attcol = jnp.sum(jnp.dot(s_exp, att, preferred_element_type=f32) * m_sel,
                     axis=-1, keepdims=True)
    agg_ref[...] = jnp.dot(s_sum, conv * attcol, preferred_element_type=f32)


def _tc_combine(nf_b, rbf_b, W1T, b1, W2T, b2, wfT, TN=128):
    N, K, G = rbf_b.shape
    F = nf_b.shape[-1] * 2
    R = TN * K
    kern = functools.partial(_combine_block, TN=TN, K=K, G=G, F=F)
    return pl.pallas_call(
        kern,
        grid=(N // TN,),
        in_specs=[
            pl.BlockSpec((R, F // 2), lambda i: (i, 0)),
            pl.BlockSpec((TN, K, G), lambda i: (i, 0, 0)),
            pl.BlockSpec((G, F), lambda i: (0, 0)),
            pl.BlockSpec((1, F), lambda i: (0, 0)),
            pl.BlockSpec((F, F), lambda i: (0, 0)),
            pl.BlockSpec((1, F), lambda i: (0, 0)),
            pl.BlockSpec((1, F), lambda i: (0, 0)),
        ],
        out_specs=[
            pl.BlockSpec((TN, F), lambda i: (i, 0)),
            pl.BlockSpec((TN, K), lambda i: (i, 0)),
        ],
        out_shape=[
            jax.ShapeDtypeStruct((N, F), jnp.float32),
            jax.ShapeDtypeStruct((N, K), jnp.float32),
        ],
        compiler_params=pltpu.CompilerParams(
            dimension_semantics=("arbitrary",)),
    )(nf_b, rbf_b, W1T, b1, W2T, b2, wfT)


@jax.jit
def _cfconv(features, rbf_expansion, neighbor_list, W1, b1, W2, b2, nbr_filter):
    B, N, F = features.shape
    _, _, K, G = rbf_expansion.shape
    fb16 = lax.bitcast_convert_type(features.astype(jnp.bfloat16),
                                    jnp.uint16).astype(jnp.uint32)
    packed = (fb16[..., F // 2:] << 16) | fb16[..., :F // 2]
    table = lax.bitcast_convert_type(packed, jnp.int32).reshape(B * N, F // 2)
    idx = (neighbor_list.astype(jnp.int32)
           + (jnp.arange(B, dtype=jnp.int32) * N)[:, None, None]
           ).reshape(B * N * K)
    rbf_flat = rbf_expansion.reshape(B * N, K, G)
    W1T = W1.T.astype(jnp.bfloat16)
    W2T = W2.T.astype(jnp.bfloat16)
    b1r, b2r, wfT = b1.reshape(1, F), b2.reshape(1, F), nbr_filter.T
    sizes = [1, 2, 2, 2, 1]
    starts = [sum(sizes[:i]) for i in range(len(sizes))]
    nfs = [_sc_gather(table, idx[s * N * K:(s + sz) * N * K])
           for s, sz in zip(starts, sizes)]
    aggs, atts = [], []
    for nf_c, s, sz in zip(nfs, starts, sizes):
        agg_c, att_c = _tc_combine(nf_c, rbf_flat[s * N:(s + sz) * N],
                                   W1T, b1r, W2T, b2r, wfT)
        aggs.append(agg_c)
        atts.append(att_c)
    return (jnp.concatenate(aggs).reshape(B, N, F),
            jnp.concatenate(atts).reshape(B, N, K))


def kernel(features, rbf_expansion, neighbor_list, W1, b1, W2, b2, nbr_filter):
    return _cfconv(features, rbf_expansion, neighbor_list,
                   W1, b1, W2, b2, nbr_filter)


# speedup vs baseline: 1.0275x; 1.0275x over previous
import functools

import jax
import jax.numpy as jnp
from jax import lax
from jax.experimental import pallas as pl
from jax.experimental.pallas import tpu as pltpu
from jax.experimental.pallas import tpu_sc as plsc

NC, NS = 2, 16
NW = NC * NS


def _sc_gather(table, idx_flat, C=128):
    RT = idx_flat.shape[0]
    F = table.shape[1]
    rows_per_w = RT // NW
    chunks_per_w = rows_per_w // C
    mesh = plsc.VectorSubcoreMesh(core_axis_name="c", subcore_axis_name="s")

    @functools.partial(
        pl.kernel, mesh=mesh,
        out_type=jax.ShapeDtypeStruct((RT, F), table.dtype),
        scratch_types=[
            pltpu.VMEM((C,), jnp.int32),
            pltpu.VMEM((C, F), table.dtype),
            pltpu.SemaphoreType.DMA,
        ],
    )
    def gk(table_hbm, idx_hbm, out_hbm, idx_v, rows_v, sem):
        wid = lax.axis_index("s") * NC + lax.axis_index("c")
        base = wid * rows_per_w

        @pl.loop(0, chunks_per_w)
        def _(ci):
            off = base + ci * C
            pltpu.sync_copy(idx_hbm.at[pl.ds(off, C)], idx_v)
            pltpu.async_copy(table_hbm.at[idx_v], rows_v, sem).wait()
            pltpu.sync_copy(rows_v, out_hbm.at[pl.ds(off, C)])

    return gk(table, idx_flat)


def _combine_block(nf_ref, rbf_ref, w1_ref, b1_ref, w2_ref, b2_ref,
                   wf_ref, agg_ref, att_ref, *, TN, K, G, F):
    R = TN * K
    f32 = jnp.float32
    rbf = rbf_ref[...].reshape(R, G)
    nrm = jnp.sqrt(jnp.sum(rbf * rbf, axis=-1, keepdims=True))
    rbfn = rbf / (nrm + 1e-8)
    h = jnp.tanh(jnp.dot(rbfn.astype(jnp.bfloat16), w1_ref[...],
                         preferred_element_type=f32) + b1_ref[...])
    cf = jnp.dot(h.astype(jnp.bfloat16), w2_ref[...],
                 preferred_element_type=f32) + b2_ref[...]

    x = nf_ref[...]
    lo = lax.bitcast_convert_type(x << 16, f32)
    hi = lax.bitcast_convert_type(x & jnp.int32(-65536), f32)
    nf = jnp.concatenate([lo, hi], axis=-1)
    conv = nf * cf
    lgcol = jnp.sum(conv * wf_ref[...], axis=-1, keepdims=True)

    jmodk = jax.lax.broadcasted_iota(jnp.int32, (R, K), 0) % K
    m_sel = (jmodk == jax.lax.broadcasted_iota(jnp.int32, (R, K), 1)).astype(f32)
    s_rows = jax.lax.broadcasted_iota(jnp.int32, (TN, R), 1) // K
    s_sum = (s_rows == jax.lax.broadcasted_iota(jnp.int32, (TN, R), 0)).astype(f32)
    e_rows = jax.lax.broadcasted_iota(jnp.int32, (R, TN), 0) // K
    s_exp = (e_rows == jax.lax.broadcasted_iota(jnp.int32, (R, TN), 1)).astype(f32)

    lg = jnp.dot(s_sum, m_sel * lgcol, preferred_element_type=f32)
    mx = jnp.max(lg, axis=-1, keepdims=True)
    ex = jnp.exp(lg - mx)
    att = ex / jnp.sum(ex, axis=-1, keepdims=True)
    att_ref[...] = att

    attcol = jnp.sum(jnp.dot(s_exp, att, preferred_element_type=f32) * m_sel,
                     axis=-1, keepdims=True)
    agg_ref[...] = jnp.dot(s_sum, conv * attcol, preferred_element_type=f32)


def _tc_combine(nf_b, rbf_b, W1T, b1, W2T, b2, wfT, TN=128):
    N, K, G = rbf_b.shape
    F = nf_b.shape[-1] * 2
    R = TN * K
    kern = functools.partial(_combine_block, TN=TN, K=K, G=G, F=F)
    return pl.pallas_call(
        kern,
        grid=(N // TN,),
        in_specs=[
            pl.BlockSpec((R, F // 2), lambda i: (i, 0)),
            pl.BlockSpec((TN, K, G), lambda i: (i, 0, 0)),
            pl.BlockSpec((G, F), lambda i: (0, 0)),
            pl.BlockSpec((1, F), lambda i: (0, 0)),
            pl.BlockSpec((F, F), lambda i: (0, 0)),
            pl.BlockSpec((1, F), lambda i: (0, 0)),
            pl.BlockSpec((1, F), lambda i: (0, 0)),
        ],
        out_specs=[
            pl.BlockSpec((TN, F), lambda i: (i, 0)),
            pl.BlockSpec((TN, K), lambda i: (i, 0)),
        ],
        out_shape=[
            jax.ShapeDtypeStruct((N, F), jnp.float32),
            jax.ShapeDtypeStruct((N, K), jnp.float32),
        ],
        compiler_params=pltpu.CompilerParams(
            dimension_semantics=("arbitrary",)),
    )(nf_b, rbf_b, W1T, b1, W2T, b2, wfT)


@jax.jit
def _cfconv(features, rbf_expansion, neighbor_list, W1, b1, W2, b2, nbr_filter):
    B, N, F = features.shape
    _, _, K, G = rbf_expansion.shape
    fb16 = lax.bitcast_convert_type(features.astype(jnp.bfloat16),
                                    jnp.uint16).astype(jnp.uint32)
    packed = (fb16[..., F // 2:] << 16) | fb16[..., :F // 2]
    table = lax.bitcast_convert_type(packed, jnp.int32).reshape(B * N, F // 2)
    idx = (neighbor_list.astype(jnp.int32)
           + (jnp.arange(B, dtype=jnp.int32) * N)[:, None, None]
           ).reshape(B * N * K)
    rbf_flat = rbf_expansion.reshape(B * N, K, G)
    W1T = W1.T.astype(jnp.bfloat16)
    W2T = W2.T.astype(jnp.bfloat16)
    b1r, b2r, wfT = b1.reshape(1, F), b2.reshape(1, F), nbr_filter.T
    sizes = [2, 2, 2, 2]
    starts = [sum(sizes[:i]) for i in range(len(sizes))]
    nfs = [_sc_gather(table, idx[s * N * K:(s + sz) * N * K])
           for s, sz in zip(starts, sizes)]
    aggs, atts = [], []
    for nf_c, s, sz in zip(nfs, starts, sizes):
        agg_c, att_c = _tc_combine(nf_c, rbf_flat[s * N:(s + sz) * N],
                                   W1T, b1r, W2T, b2r, wfT)
        aggs.append(agg_c)
        atts.append(att_c)
    return (jnp.concatenate(aggs).reshape(B, N, F),
            jnp.concatenate(atts).reshape(B, N, K))


def kernel(features, rbf_expansion, neighbor_list, W1, b1, W2, b2, nbr_filter):
    return _cfconv(features, rbf_expansion, neighbor_list,
                   W1, b1, W2, b2, nbr_filter)
